# Initial kernel scaffold; baseline (speedup 1.0000x reference)
import jax
import jax.numpy as jnp
from jax.experimental import pallas as pl


def kernel(h, edge_index, edge_w, W1, W2, b2):
    raise NotImplementedError("write your pallas kernel here")


# speedup vs baseline: 2.3114x; 2.3114x over previous
import functools

import jax
import jax.numpy as jnp
from jax import lax
from jax.experimental import pallas as pl
from jax.experimental.pallas import tpu as pltpu
from jax.experimental.pallas import tpu_sc as plsc

F = 128
C = 128
NSUB = 16
NCORE = 2
NW = NSUB * NCORE
RB = 64
TCB = 512


def _tc1_body(h_ref, w1a_ref, w2a_ref, b2_ref, z_ref, p_ref):
    hb = h_ref[...]
    dn = (((1,), (1,)), ((), ()))
    z_ref[...] = lax.dot_general(hb, w1a_ref[...], dn,
                                 preferred_element_type=jnp.float32)
    p_ref[...] = lax.dot_general(hb, w2a_ref[...], dn,
                                 preferred_element_type=jnp.float32) + b2_ref[...]


def _tc2_body(p_ref, hn_ref, w2b_ref, o_ref):
    a = hn_ref[0] + hn_ref[1]
    dn = (((1,), (1,)), ((), ()))
    o = p_ref[...] + lax.dot_general(a, w2b_ref[...], dn,
                                     preferred_element_type=jnp.float32)
    o_ref[...] = jnp.maximum(o, 0.0)


def _sc_body(npad, ept, z_hbm, src_hbm, dst_hbm, ew_hbm, w1b_hbm, out_hbm,
             sidx, didxa, didxb, ew_v, zrows, w1b_v, abuf, cntbuf, ones_v,
             acc_sp, cnt_sp, sem):
    rows_per_tile = npad // NSUB
    chunks = ept // C
    cnt_chunks = (ept * NW) // NSUB // C
    cid = lax.axis_index("c")
    sid = lax.axis_index("s")
    wid = cid * NSUB + sid
    row0 = sid * rows_per_tile

    pltpu.sync_copy(w1b_hbm, w1b_v)
    zeros16 = jnp.zeros((16,), jnp.float32)
    ones16 = jnp.ones((16,), jnp.float32)
    for i in range(C // 16):
        ones_v[pl.ds(16 * i, 16)] = ones16

    def zb(r, _):
        for f in range(F // 16):
            abuf[r, pl.ds(16 * f, 16)] = zeros16
        return 0
    lax.fori_loop(0, RB, zb, 0)

    def zc(i, _):
        cntbuf[pl.ds(16 * i, 16)] = zeros16
        return 0
    lax.fori_loop(0, rows_per_tile // 16, zc, 0)

    def zs(b, _):
        pltpu.sync_copy(abuf, acc_sp.at[pl.ds(row0 + RB * b, RB)])
        return 0
    lax.fori_loop(0, rows_per_tile // RB, zs, 0)
    pltpu.sync_copy(cntbuf.at[pl.ds(0, rows_per_tile)],
                    cnt_sp.at[pl.ds(row0, rows_per_tile)])
    plsc.subcore_barrier()

    def cb(q, _):
        base = sid * (cnt_chunks * C) + q * C
        pltpu.sync_copy(dst_hbm.at[pl.ds(base, C)], didxa)
        pltpu.sync_copy(ones_v, cnt_sp.at[didxa], add=True)
        return 0
    lax.fori_loop(0, cnt_chunks, cb, 0)

    bv = [[w1b_v[j, pl.ds(16 * f, 16)] for f in range(F // 16)]
          for j in range(3)]

    def mb(g, _):
        base = wid * ept + g * C
        pltpu.sync_copy(src_hbm.at[pl.ds(base, C)], sidx)
        pltpu.sync_copy(dst_hbm.at[pl.ds(base, C)], didxb)
        pltpu.sync_copy(ew_hbm.at[pl.ds(3 * base, 3 * C)], ew_v.at[pl.ds(0, 3 * C)])
        pltpu.async_copy(z_hbm.at[sidx], zrows, sem).wait()

        def eb(e, _):
            wv = ew_v[pl.ds(3 * e, 16)]
            w0 = wv[0]
            w1 = wv[1]
            w2 = wv[2]
            for f in range(F // 16):
                sl = pl.ds(16 * f, 16)
                x = zrows[e, sl] + w0 * bv[0][f] + w1 * bv[1][f] + w2 * bv[2][f]
                zrows[e, sl] = jnp.maximum(x, 0.01 * x)
            return 0
        lax.fori_loop(0, C, eb, 0)
        pltpu.sync_copy(zrows, acc_sp.at[didxb], add=True)
        return 0
    lax.fori_loop(0, chunks, mb, 0)
    plsc.subcore_barrier()

    pltpu.sync_copy(cnt_sp.at[pl.ds(row0, rows_per_tile)],
                    cntbuf.at[pl.ds(0, rows_per_tile)])

    def rcp(i, _):
        sl = pl.ds(16 * i, 16)
        cntbuf[sl] = 1.0 / jnp.maximum(cntbuf[sl], 1.0)
        return 0
    lax.fori_loop(0, rows_per_tile // 16, rcp, 0)

    def ob(b, _):
        r0 = row0 + RB * b
        pltpu.sync_copy(acc_sp.at[pl.ds(r0, RB)], abuf)

        def sb(r, _):
            s = cntbuf[pl.ds(RB * b + r, 16)][0]
            for f in range(F // 16):
                sl = pl.ds(16 * f, 16)
                abuf[r, sl] = abuf[r, sl] * s
            return 0
        lax.fori_loop(0, RB, sb, 0)
        pltpu.sync_copy(abuf, out_hbm.at[cid, pl.ds(r0, RB)])
        return 0
    lax.fori_loop(0, rows_per_tile // RB, ob, 0)


def kernel(h, edge_index, edge_w, W1, W2, b2):
    n = h.shape[0]
    e = edge_index.shape[1]
    npad = ((n + TCB - 1) // TCB) * TCB
    ept = ((e + NW * C - 1) // (NW * C)) * C
    etot = ept * NW

    src = edge_index[0].astype(jnp.int32)
    dst = edge_index[1].astype(jnp.int32)
    src_p = jnp.concatenate([src, jnp.zeros((etot - e,), jnp.int32)])
    dst_p = jnp.concatenate([dst, jnp.full((etot - e,), n, jnp.int32)])
    ew_p = jnp.concatenate([edge_w, jnp.zeros((etot - e, 3), jnp.float32)]
                           ).reshape(-1)
    h_p = jnp.pad(h, ((0, npad - n), (0, 0)))
    W1a = W1[:, :F]
    w1bT = jnp.transpose(W1[:, F:])
    W2a = W2[:, :F]
    W2b = W2[:, F:]
    b2r = b2.reshape(1, F)

    grid = (npad // TCB,)
    Z, P = pl.pallas_call(
        _tc1_body,
        grid=grid,
        in_specs=[
            pl.BlockSpec((TCB, F), lambda i: (i, 0)),
            pl.BlockSpec((F, F), lambda i: (0, 0)),
            pl.BlockSpec((F, F), lambda i: (0, 0)),
            pl.BlockSpec((1, F), lambda i: (0, 0)),
        ],
        out_specs=[pl.BlockSpec((TCB, F), lambda i: (i, 0)),
                   pl.BlockSpec((TCB, F), lambda i: (i, 0))],
        out_shape=[jax.ShapeDtypeStruct((npad, F), jnp.float32),
                   jax.ShapeDtypeStruct((npad, F), jnp.float32)],
    )(h_p, W1a, W2a, b2r)

    mesh = plsc.VectorSubcoreMesh(core_axis_name="c", subcore_axis_name="s")
    hn = pl.kernel(
        functools.partial(_sc_body, npad, ept),
        out_type=jax.ShapeDtypeStruct((NCORE, npad, F), jnp.float32),
        mesh=mesh,
        scratch_types=[
            pltpu.VMEM((C,), jnp.int32),
            pltpu.VMEM((C,), jnp.int32),
            pltpu.VMEM((C,), jnp.int32),
            pltpu.VMEM((3 * C + 16,), jnp.float32),
            pltpu.VMEM((C, F), jnp.float32),
            pltpu.VMEM((3, F), jnp.float32),
            pltpu.VMEM((RB, F), jnp.float32),
            pltpu.VMEM((npad // NSUB + 16,), jnp.float32),
            pltpu.VMEM((C,), jnp.float32),
            pltpu.VMEM_SHARED((npad, F), jnp.float32),
            pltpu.VMEM_SHARED((npad,), jnp.float32),
            pltpu.SemaphoreType.DMA,
        ],
    )(Z, src_p, dst_p, ew_p, w1bT)

    out = pl.pallas_call(
        _tc2_body,
        grid=grid,
        in_specs=[
            pl.BlockSpec((TCB, F), lambda i: (i, 0)),
            pl.BlockSpec((NCORE, TCB, F), lambda i: (0, i, 0)),
            pl.BlockSpec((F, F), lambda i: (0, 0)),
        ],
        out_specs=pl.BlockSpec((TCB, F), lambda i: (i, 0)),
        out_shape=jax.ShapeDtypeStruct((npad, F), jnp.float32),
    )(P, hn, W2b)
    return out[:n]
